# Initial kernel scaffold; baseline (speedup 1.0000x reference)
import jax
import jax.numpy as jnp
from jax.experimental import pallas as pl


def kernel(voxel_features, voxel_xyz_indices, num_valid_voxels, W0, W1, NW0, NW1, gamma0, beta0, gamma1, beta1):
    raise NotImplementedError("write your pallas kernel here")


# speedup vs baseline: 2.6348x; 2.6348x over previous
import functools

import jax
import jax.numpy as jnp
from jax import lax
from jax.experimental import pallas as pl
from jax.experimental.pallas import tpu as pltpu
from jax.experimental.pallas import tpu_sc as plsc

EPS = 1e-3
B, N, C = 4, 2048, 32
M = B * N
NC, NS = 2, 16
NR = 13824
ZCH = NR // NS
VPW = 128
NW_ = NC * NS * NC
SCH = 16
GCH = 32
MB = 1024


def _make_sc_kernel():
    mesh = plsc.VectorSubcoreMesh(core_axis_name="c", subcore_axis_name="s")

    @functools.partial(
        pl.kernel,
        mesh=mesh,
        out_type=jax.ShapeDtypeStruct((9, M, 128), jnp.float32),
        scratch_types=[
            pltpu.VMEM_SHARED((NR, 128), jnp.float32),
            pltpu.VMEM((16, 128), jnp.float32),
            pltpu.VMEM((2, SCH, 128), jnp.float32),
            pltpu.VMEM((24, SCH), jnp.int32),
            pltpu.VMEM((9, VPW), jnp.int32),
            pltpu.VMEM((2, GCH, 128), jnp.float32),
            pltpu.SemaphoreType.DMA,
            pltpu.SemaphoreType.DMA,
            pltpu.SemaphoreType.DMA,
        ],
    )
    def sc_kernel(upd_hbm, sidx_hbm, gidx_hbm, zeros_hbm, lhs_hbm,
                  grid_sh, zbuf, upd_v, sidx_v, gidx_v, gbuf,
                  sem_a, sem_g, sem_o):
        c = lax.axis_index("c")
        s = lax.axis_index("s")
        pltpu.sync_copy(zeros_hbm, zbuf)
        for rnd in range(NC):
            ex = c * 2 + rnd
            v0 = pl.multiple_of(ex * N + s * VPW, VPW)
            w = ex * NS + s

            z0 = pl.multiple_of(s * ZCH, 8)
            zcps = []
            for j in range(ZCH // 16):
                zcps.append(pltpu.async_copy(
                    zbuf, grid_sh.at[pl.ds(z0 + j * 16, 16)], sem_a))
            for cp in zcps:
                cp.wait()
            plsc.subcore_barrier()

            pltpu.sync_copy(sidx_hbm.at[w], sidx_v)
            scps = []
            for k in range(24):
                h = k % 2
                if k >= 2:
                    scps[k - 2].wait()
                pltpu.sync_copy(
                    upd_hbm.at[k // 8, pl.ds(v0 + (k % 8) * SCH, SCH)],
                    upd_v.at[h])
                scps.append(pltpu.async_copy(
                    upd_v.at[h], grid_sh.at[sidx_v.at[k]], sem_a, add=True))
            for cp in scps[-2:]:
                cp.wait()
            plsc.subcore_barrier()

            pltpu.sync_copy(gidx_hbm.at[w], gidx_v)
            ocps = []
            for k in range(9 * (VPW // GCH)):
                o, q = k // (VPW // GCH), k % (VPW // GCH)
                h = k % 2
                if k >= 2:
                    ocps[k - 2].wait()
                pltpu.async_copy(
                    grid_sh.at[gidx_v.at[o, pl.ds(q * GCH, GCH)]],
                    gbuf.at[h], sem_g).wait()
                ocps.append(pltpu.async_copy(
                    gbuf.at[h],
                    lhs_hbm.at[o, pl.ds(v0 + q * GCH, GCH)], sem_o))
            for cp in ocps[-2:]:
                cp.wait()
            plsc.subcore_barrier()

    return sc_kernel


_SC_KERNEL = _make_sc_kernel()


def _tc_body(nv_ref, lhs_ref, msk_ref, w_ref, par_ref, out_ref, acc_ref):
    i = pl.program_id(0)

    @pl.when(i == 0)
    def _init():
        acc_ref[...] = jnp.zeros_like(acc_ref)

    @pl.when(i < M // MB)
    def _matmul():
        x64 = jnp.zeros((MB, 64), jnp.float32)
        for o in range(9):
            xo = lhs_ref[o] * msk_ref[:, o:o + 1]
            x64 = x64 + jnp.dot(xo, w_ref[o],
                                preferred_element_type=jnp.float32)
        x = x64[:, :32] / (1.0 + jnp.abs(x64[:, 32:33]))
        b = i // (N // MB)
        nv = nv_ref[b]
        n_local = (lax.broadcasted_iota(jnp.int32, (MB, 1), 0)
                   + (i % (N // MB)) * MB)
        m = (n_local < nv).astype(jnp.float32)
        xm = x * m
        out_ref[pl.ds(i * MB, MB), :] = xm
        acc_ref[0:1, :32] += jnp.sum(xm, axis=0, keepdims=True)
        acc_ref[1:2, :32] += jnp.sum(xm * xm, axis=0, keepdims=True)

    @pl.when(i == M // MB)
    def _bn():
        count = nv_ref[0] + nv_ref[1] + nv_ref[2] + nv_ref[3]
        cnt = jnp.maximum(count.astype(jnp.float32), 1.0)
        mean = acc_ref[0:1, :32] / cnt
        var = acc_ref[1:2, :32] / cnt - mean * mean
        scale = lax.rsqrt(var + EPS) * par_ref[0:1, :32]
        riota = lax.broadcasted_iota(jnp.int32, (M, 1), 0)
        bidx = riota // N
        nvrow = jnp.where(bidx == 0, nv_ref[0],
                          jnp.where(bidx == 1, nv_ref[1],
                                    jnp.where(bidx == 2, nv_ref[2],
                                              nv_ref[3])))
        m = ((riota - bidx * N) < nvrow).astype(jnp.float32)
        xm = out_ref[...]
        y = ((xm - mean) * scale + par_ref[1:2, :32]) * m
        out_ref[...] = jnp.maximum(y, 0.0)


def _tc_layer(nv, lhs, msk, wext, par):
    return pl.pallas_call(
        _tc_body,
        grid=(M // MB + 1,),
        in_specs=[
            pl.BlockSpec(memory_space=pltpu.SMEM),
            pl.BlockSpec((9, MB, 128),
                         lambda i: (0, jnp.minimum(i, M // MB - 1), 0)),
            pl.BlockSpec((MB, 16),
                         lambda i: (jnp.minimum(i, M // MB - 1), 0)),
            pl.BlockSpec((9, 128, 64), lambda i: (0, 0, 0)),
            pl.BlockSpec((8, 128), lambda i: (0, 0)),
        ],
        out_specs=pl.BlockSpec((M, 32), lambda i: (0, 0)),
        out_shape=jax.ShapeDtypeStruct((M, 32), jnp.float32),
        scratch_shapes=[pltpu.VMEM((8, 128), jnp.float32)],
    )(nv, lhs, msk, wext, par)


def kernel(voxel_features, voxel_xyz_indices, num_valid_voxels,
           W0, W1, NW0, NW1, gamma0, beta0, gamma1, beta1):
    coords = voxel_xyz_indices.reshape(M, 3).astype(jnp.int32)
    b_all = jnp.arange(M, dtype=jnp.int32) // N
    n_all = jnp.arange(M, dtype=jnp.int32) % N
    valid = n_all < num_valid_voxels[b_all]
    x, y, z = coords[:, 0], coords[:, 1], coords[:, 2]
    r = x * 576 + y * 24 + z

    ok_a = valid & (z > 0)
    ok_c = valid & (z < 23)
    sidx = jnp.stack([jnp.clip(r - 1, 0, NR - 1), r,
                      jnp.clip(r + 1, 0, NR - 1)]).astype(jnp.int32)
    sidx = sidx.reshape(3, NW_, VPW // SCH, SCH).transpose(1, 0, 2, 3)
    sidx = sidx.reshape(NW_, 24, SCH)

    gcols, mcols = [], []
    for dx in (-1, 0, 1):
        for dy in (-1, 0, 1):
            xq, yq = x + dx, y + dy
            oob = (xq < 0) | (xq > 23) | (yq < 0) | (yq > 23)
            gcols.append(jnp.where(oob, r, xq * 576 + yq * 24 + z))
            mcols.append(jnp.where(oob, 0.0, 1.0))
    gidx = jnp.stack(gcols, axis=1).astype(jnp.int32)
    gidx = gidx.reshape(NW_, VPW, 9).transpose(0, 2, 1)
    msk = jnp.stack(mcols, axis=1)
    msk = jnp.pad(msk, ((0, 0), (0, 7)))

    zeros = jnp.zeros((16, 128), jnp.float32)
    z32 = jnp.zeros((M, 32), jnp.float32)
    onem = valid.astype(jnp.float32)[:, None]
    cnt_a = jnp.zeros((M, 32), jnp.float32).at[:, 2].set(1.0)
    cnt_b = jnp.zeros((M, 32), jnp.float32).at[:, 1].set(1.0)
    cnt_c = jnp.zeros((M, 32), jnp.float32).at[:, 0].set(1.0)
    m_a = (ok_a.astype(jnp.float32))[:, None]
    m_b = onem
    m_c = (ok_c.astype(jnp.float32))[:, None]

    def wext(W, NW):
        we = jnp.zeros((9, 128, 64), jnp.float32)
        we = we.at[:, :96, :32].set(W.reshape(9, 96, 32))
        we = we.at[:, 96:99, 32].set(NW.reshape(9, 3))
        return we

    def par(gamma, beta):
        p = jnp.zeros((8, 128), jnp.float32)
        p = p.at[0, :32].set(gamma)
        p = p.at[1, :32].set(beta)
        return p

    nv = num_valid_voxels.astype(jnp.int32)

    net = voxel_features.reshape(M, C)
    for we, pp in ((wext(W0, NW0), par(gamma0, beta0)),
                   (wext(W1, NW1), par(gamma1, beta1))):
        upd = jnp.stack([
            jnp.concatenate([z32, z32, net, cnt_a], axis=1) * m_a,
            jnp.concatenate([z32, net, z32, cnt_b], axis=1) * m_b,
            jnp.concatenate([net, z32, z32, cnt_c], axis=1) * m_c,
        ])
        lhs = _SC_KERNEL(upd, sidx, gidx, zeros)
        net = _tc_layer(nv, lhs, msk, we, pp)
    return net.reshape(B, N, C)


# speedup vs baseline: 2.7524x; 1.0446x over previous
import functools

import jax
import jax.numpy as jnp
from jax import lax
from jax.experimental import pallas as pl
from jax.experimental.pallas import tpu as pltpu
from jax.experimental.pallas import tpu_sc as plsc

EPS = 1e-3
B, N, C = 4, 2048, 32
M = B * N
NC, NS = 2, 16
NR = 13824
ZCH = NR // NS
VPW = 128
NW_ = NC * NS * NC
SCH = 16
GCH = 32
MB = 1024


def _make_sc_kernel():
    mesh = plsc.VectorSubcoreMesh(core_axis_name="c", subcore_axis_name="s")

    @functools.partial(
        pl.kernel,
        mesh=mesh,
        out_type=jax.ShapeDtypeStruct((9, M, 128), jnp.float32),
        scratch_types=[
            pltpu.VMEM_SHARED((NR, 128), jnp.float32),
            pltpu.VMEM((16, 128), jnp.float32),
            pltpu.VMEM((2, SCH, 128), jnp.float32),
            pltpu.VMEM((24, SCH), jnp.int32),
            pltpu.VMEM((9, VPW), jnp.int32),
            pltpu.VMEM((2, GCH, 128), jnp.float32),
            pltpu.SemaphoreType.DMA,
            pltpu.SemaphoreType.DMA,
            pltpu.SemaphoreType.DMA,
        ],
    )
    def sc_kernel(upd_hbm, sidx_hbm, gidx_hbm, zeros_hbm, lhs_hbm,
                  grid_sh, zbuf, upd_v, sidx_v, gidx_v, gbuf,
                  sem_a, sem_g, sem_o):
        c = lax.axis_index("c")
        s = lax.axis_index("s")
        pltpu.sync_copy(zeros_hbm, zbuf)
        for rnd in range(NC):
            ex = c * 2 + rnd
            v0 = pl.multiple_of(ex * N + s * VPW, VPW)
            w = ex * NS + s

            z0 = pl.multiple_of(s * ZCH, 8)
            zcps = []
            for j in range(ZCH // 16):
                zcps.append(pltpu.async_copy(
                    zbuf, grid_sh.at[pl.ds(z0 + j * 16, 16)], sem_a))
            for cp in zcps:
                cp.wait()
            plsc.subcore_barrier()

            pltpu.sync_copy(sidx_hbm.at[w], sidx_v)
            scps = []
            for k in range(24):
                h = k % 2
                if k >= 2:
                    scps[k - 2].wait()
                pltpu.sync_copy(
                    upd_hbm.at[k // 8, pl.ds(v0 + (k % 8) * SCH, SCH)],
                    upd_v.at[h])
                scps.append(pltpu.async_copy(
                    upd_v.at[h], grid_sh.at[sidx_v.at[k]], sem_a, add=True))
            for cp in scps[-2:]:
                cp.wait()
            plsc.subcore_barrier()

            pltpu.sync_copy(gidx_hbm.at[w], gidx_v)
            nch = 9 * (VPW // GCH)

            def _gissue(k):
                o, q = k // (VPW // GCH), k % (VPW // GCH)
                return pltpu.async_copy(
                    grid_sh.at[gidx_v.at[o, pl.ds(q * GCH, GCH)]],
                    gbuf.at[k % 2], sem_g)

            gcps = {0: _gissue(0)}
            ocps = {}
            for k in range(nch):
                o, q = k // (VPW // GCH), k % (VPW // GCH)
                if k + 1 < nch:
                    if k >= 1:
                        ocps[k - 1].wait()
                    gcps[k + 1] = _gissue(k + 1)
                gcps[k].wait()
                ocps[k] = pltpu.async_copy(
                    gbuf.at[k % 2],
                    lhs_hbm.at[o, pl.ds(v0 + q * GCH, GCH)], sem_o)
            ocps[nch - 1].wait()
            plsc.subcore_barrier()

    return sc_kernel


_SC_KERNEL = _make_sc_kernel()


def _tc_body(nv_ref, lhs_ref, msk_ref, w_ref, par_ref, out_ref, acc_ref):
    i = pl.program_id(0)

    @pl.when(i == 0)
    def _init():
        acc_ref[...] = jnp.zeros_like(acc_ref)

    @pl.when(i < M // MB)
    def _matmul():
        x64 = jnp.zeros((MB, 64), jnp.float32)
        for o in range(9):
            xo = lhs_ref[o] * msk_ref[:, o:o + 1]
            x64 = x64 + jnp.dot(xo, w_ref[o],
                                preferred_element_type=jnp.float32)
        x = x64[:, :32] / (1.0 + jnp.abs(x64[:, 32:33]))
        b = i // (N // MB)
        nv = nv_ref[b]
        n_local = (lax.broadcasted_iota(jnp.int32, (MB, 1), 0)
                   + (i % (N // MB)) * MB)
        m = (n_local < nv).astype(jnp.float32)
        xm = x * m
        out_ref[pl.ds(i * MB, MB), :] = xm
        acc_ref[0:1, :32] += jnp.sum(xm, axis=0, keepdims=True)
        acc_ref[1:2, :32] += jnp.sum(xm * xm, axis=0, keepdims=True)

    @pl.when(i == M // MB)
    def _bn():
        count = nv_ref[0] + nv_ref[1] + nv_ref[2] + nv_ref[3]
        cnt = jnp.maximum(count.astype(jnp.float32), 1.0)
        mean = acc_ref[0:1, :32] / cnt
        var = acc_ref[1:2, :32] / cnt - mean * mean
        scale = lax.rsqrt(var + EPS) * par_ref[0:1, :32]
        riota = lax.broadcasted_iota(jnp.int32, (M, 1), 0)
        bidx = riota // N
        nvrow = jnp.where(bidx == 0, nv_ref[0],
                          jnp.where(bidx == 1, nv_ref[1],
                                    jnp.where(bidx == 2, nv_ref[2],
                                              nv_ref[3])))
        m = ((riota - bidx * N) < nvrow).astype(jnp.float32)
        xm = out_ref[...]
        y = ((xm - mean) * scale + par_ref[1:2, :32]) * m
        out_ref[...] = jnp.maximum(y, 0.0)


def _tc_layer(nv, lhs, msk, wext, par):
    return pl.pallas_call(
        _tc_body,
        grid=(M // MB + 1,),
        in_specs=[
            pl.BlockSpec(memory_space=pltpu.SMEM),
            pl.BlockSpec((9, MB, 128),
                         lambda i: (0, jnp.minimum(i, M // MB - 1), 0)),
            pl.BlockSpec((MB, 16),
                         lambda i: (jnp.minimum(i, M // MB - 1), 0)),
            pl.BlockSpec((9, 128, 64), lambda i: (0, 0, 0)),
            pl.BlockSpec((8, 128), lambda i: (0, 0)),
        ],
        out_specs=pl.BlockSpec((M, 32), lambda i: (0, 0)),
        out_shape=jax.ShapeDtypeStruct((M, 32), jnp.float32),
        scratch_shapes=[pltpu.VMEM((8, 128), jnp.float32)],
    )(nv, lhs, msk, wext, par)


def kernel(voxel_features, voxel_xyz_indices, num_valid_voxels,
           W0, W1, NW0, NW1, gamma0, beta0, gamma1, beta1):
    coords = voxel_xyz_indices.reshape(M, 3).astype(jnp.int32)
    b_all = jnp.arange(M, dtype=jnp.int32) // N
    n_all = jnp.arange(M, dtype=jnp.int32) % N
    valid = n_all < num_valid_voxels[b_all]
    x, y, z = coords[:, 0], coords[:, 1], coords[:, 2]
    r = x * 576 + y * 24 + z

    ok_a = valid & (z > 0)
    ok_c = valid & (z < 23)
    sidx = jnp.stack([jnp.clip(r - 1, 0, NR - 1), r,
                      jnp.clip(r + 1, 0, NR - 1)]).astype(jnp.int32)
    sidx = sidx.reshape(3, NW_, VPW // SCH, SCH).transpose(1, 0, 2, 3)
    sidx = sidx.reshape(NW_, 24, SCH)

    gcols, mcols = [], []
    for dx in (-1, 0, 1):
        for dy in (-1, 0, 1):
            xq, yq = x + dx, y + dy
            oob = (xq < 0) | (xq > 23) | (yq < 0) | (yq > 23)
            gcols.append(jnp.where(oob, r, xq * 576 + yq * 24 + z))
            mcols.append(jnp.where(oob, 0.0, 1.0))
    gidx = jnp.stack(gcols, axis=1).astype(jnp.int32)
    gidx = gidx.reshape(NW_, VPW, 9).transpose(0, 2, 1)
    msk = jnp.stack(mcols, axis=1)
    msk = jnp.pad(msk, ((0, 0), (0, 7)))

    zeros = jnp.zeros((16, 128), jnp.float32)
    z32 = jnp.zeros((M, 32), jnp.float32)
    onem = valid.astype(jnp.float32)[:, None]
    cnt_a = jnp.zeros((M, 32), jnp.float32).at[:, 2].set(1.0)
    cnt_b = jnp.zeros((M, 32), jnp.float32).at[:, 1].set(1.0)
    cnt_c = jnp.zeros((M, 32), jnp.float32).at[:, 0].set(1.0)
    m_a = (ok_a.astype(jnp.float32))[:, None]
    m_b = onem
    m_c = (ok_c.astype(jnp.float32))[:, None]

    def wext(W, NW):
        we = jnp.zeros((9, 128, 64), jnp.float32)
        we = we.at[:, :96, :32].set(W.reshape(9, 96, 32))
        we = we.at[:, 96:99, 32].set(NW.reshape(9, 3))
        return we

    def par(gamma, beta):
        p = jnp.zeros((8, 128), jnp.float32)
        p = p.at[0, :32].set(gamma)
        p = p.at[1, :32].set(beta)
        return p

    nv = num_valid_voxels.astype(jnp.int32)

    net = voxel_features.reshape(M, C)
    for we, pp in ((wext(W0, NW0), par(gamma0, beta0)),
                   (wext(W1, NW1), par(gamma1, beta1))):
        upd = jnp.stack([
            jnp.concatenate([z32, z32, net, cnt_a], axis=1) * m_a,
            jnp.concatenate([z32, net, z32, cnt_b], axis=1) * m_b,
            jnp.concatenate([net, z32, z32, cnt_c], axis=1) * m_c,
        ])
        lhs = _SC_KERNEL(upd, sidx, gidx, zeros)
        net = _tc_layer(nv, lhs, msk, we, pp)
    return net.reshape(B, N, C)


# speedup vs baseline: 2.9355x; 1.0665x over previous
import functools

import jax
import jax.numpy as jnp
from jax import lax
from jax.experimental import pallas as pl
from jax.experimental.pallas import tpu as pltpu
from jax.experimental.pallas import tpu_sc as plsc

EPS = 1e-3
B, N, C = 4, 2048, 32
M = B * N
NC, NS = 2, 16
NR = 13824
ZCH = NR // NS
VPW = 128
NW_ = NC * NS * NC
SCH = 16
GCH = 32
MB = 1024


def _make_sc_kernel():
    mesh = plsc.VectorSubcoreMesh(core_axis_name="c", subcore_axis_name="s")

    @functools.partial(
        pl.kernel,
        mesh=mesh,
        out_type=jax.ShapeDtypeStruct((9, M, 128), jnp.float32),
        scratch_types=[
            pltpu.VMEM_SHARED((NR, 128), jnp.float32),
            pltpu.VMEM((16, 128), jnp.float32),
            pltpu.VMEM((2, SCH, 128), jnp.float32),
            pltpu.VMEM((24, SCH), jnp.int32),
            pltpu.VMEM((9, VPW), jnp.int32),
            pltpu.VMEM((2, GCH, 128), jnp.float32),
            pltpu.SemaphoreType.DMA,
            pltpu.SemaphoreType.DMA,
            pltpu.SemaphoreType.DMA,
        ],
    )
    def sc_kernel(upd_hbm, sidx_hbm, gidx_hbm, zeros_hbm, lhs_hbm,
                  grid_sh, zbuf, upd_v, sidx_v, gidx_v, gbuf,
                  sem_a, sem_g, sem_o):
        c = lax.axis_index("c")
        s = lax.axis_index("s")
        pltpu.sync_copy(zeros_hbm, zbuf)
        for rnd in range(NC):
            ex = c * 2 + rnd
            v0 = pl.multiple_of(ex * N + s * VPW, VPW)
            w = ex * NS + s

            z0 = pl.multiple_of(s * ZCH, 8)
            zcps = []
            for j in range(ZCH // 16):
                zcps.append(pltpu.async_copy(
                    zbuf, grid_sh.at[pl.ds(z0 + j * 16, 16)], sem_a))
            for cp in zcps:
                cp.wait()
            plsc.subcore_barrier()

            pltpu.sync_copy(sidx_hbm.at[w], sidx_v)

            def _sissue(k):
                return pltpu.async_copy(
                    upd_hbm.at[k // 8, pl.ds(v0 + (k % 8) * SCH, SCH)],
                    upd_v.at[k % 2], sem_g)

            stcps = {0: _sissue(0)}
            scps = {}
            for k in range(24):
                if k + 1 < 24:
                    if k >= 1:
                        scps[k - 1].wait()
                    stcps[k + 1] = _sissue(k + 1)
                stcps[k].wait()
                scps[k] = pltpu.async_copy(
                    upd_v.at[k % 2], grid_sh.at[sidx_v.at[k]], sem_a,
                    add=True)
            scps[22].wait()
            scps[23].wait()
            plsc.subcore_barrier()

            pltpu.sync_copy(gidx_hbm.at[w], gidx_v)
            nch = 9 * (VPW // GCH)

            def _gissue(k):
                o, q = k // (VPW // GCH), k % (VPW // GCH)
                return pltpu.async_copy(
                    grid_sh.at[gidx_v.at[o, pl.ds(q * GCH, GCH)]],
                    gbuf.at[k % 2], sem_g)

            gcps = {0: _gissue(0)}
            ocps = {}
            for k in range(nch):
                o, q = k // (VPW // GCH), k % (VPW // GCH)
                if k + 1 < nch:
                    if k >= 1:
                        ocps[k - 1].wait()
                    gcps[k + 1] = _gissue(k + 1)
                gcps[k].wait()
                ocps[k] = pltpu.async_copy(
                    gbuf.at[k % 2],
                    lhs_hbm.at[o, pl.ds(v0 + q * GCH, GCH)], sem_o)
            ocps[nch - 2].wait()
            ocps[nch - 1].wait()
            plsc.subcore_barrier()

    return sc_kernel


_SC_KERNEL = _make_sc_kernel()


def _tc_body(nv_ref, lhs_ref, msk_ref, w_ref, par_ref, out_ref, acc_ref):
    i = pl.program_id(0)

    @pl.when(i == 0)
    def _init():
        acc_ref[...] = jnp.zeros_like(acc_ref)

    @pl.when(i < M // MB)
    def _matmul():
        acc = jnp.zeros((MB, 128), jnp.float32)
        for o in range(9):
            xo = (lhs_ref[o] * msk_ref[:, o:o + 1]).astype(jnp.bfloat16)
            acc = acc + jnp.dot(xo, w_ref[o],
                                preferred_element_type=jnp.float32)
        x64 = acc[:, :64] + acc[:, 64:]
        x = x64[:, :32] / (1.0 + jnp.abs(x64[:, 32:33]))
        b = i // (N // MB)
        nv = nv_ref[b]
        n_local = (lax.broadcasted_iota(jnp.int32, (MB, 1), 0)
                   + (i % (N // MB)) * MB)
        m = (n_local < nv).astype(jnp.float32)
        xm = x * m
        out_ref[pl.ds(i * MB, MB), :] = xm
        acc_ref[0:1, :32] += jnp.sum(xm, axis=0, keepdims=True)
        acc_ref[1:2, :32] += jnp.sum(xm * xm, axis=0, keepdims=True)

    @pl.when(i == M // MB)
    def _bn():
        count = nv_ref[0] + nv_ref[1] + nv_ref[2] + nv_ref[3]
        cnt = jnp.maximum(count.astype(jnp.float32), 1.0)
        mean = acc_ref[0:1, :32] / cnt
        var = acc_ref[1:2, :32] / cnt - mean * mean
        scale = lax.rsqrt(var + EPS) * par_ref[0:1, :32]
        riota = lax.broadcasted_iota(jnp.int32, (M, 1), 0)
        bidx = riota // N
        nvrow = jnp.where(bidx == 0, nv_ref[0],
                          jnp.where(bidx == 1, nv_ref[1],
                                    jnp.where(bidx == 2, nv_ref[2],
                                              nv_ref[3])))
        m = ((riota - bidx * N) < nvrow).astype(jnp.float32)
        xm = out_ref[...]
        y = ((xm - mean) * scale + par_ref[1:2, :32]) * m
        out_ref[...] = jnp.maximum(y, 0.0)


def _tc_layer(nv, lhs, msk, wext, par):
    return pl.pallas_call(
        _tc_body,
        grid=(M // MB + 1,),
        in_specs=[
            pl.BlockSpec(memory_space=pltpu.SMEM),
            pl.BlockSpec((9, MB, 128),
                         lambda i: (0, jnp.minimum(i, M // MB - 1), 0)),
            pl.BlockSpec((MB, 16),
                         lambda i: (jnp.minimum(i, M // MB - 1), 0)),
            pl.BlockSpec((9, 128, 128), lambda i: (0, 0, 0)),
            pl.BlockSpec((8, 128), lambda i: (0, 0)),
        ],
        out_specs=pl.BlockSpec((M, 32), lambda i: (0, 0)),
        out_shape=jax.ShapeDtypeStruct((M, 32), jnp.float32),
        scratch_shapes=[pltpu.VMEM((8, 128), jnp.float32)],
    )(nv, lhs, msk, wext, par)


def kernel(voxel_features, voxel_xyz_indices, num_valid_voxels,
           W0, W1, NW0, NW1, gamma0, beta0, gamma1, beta1):
    coords = voxel_xyz_indices.reshape(M, 3).astype(jnp.int32)
    b_all = jnp.arange(M, dtype=jnp.int32) // N
    n_all = jnp.arange(M, dtype=jnp.int32) % N
    valid = n_all < num_valid_voxels[b_all]
    x, y, z = coords[:, 0], coords[:, 1], coords[:, 2]
    r = x * 576 + y * 24 + z

    ok_a = valid & (z > 0)
    ok_c = valid & (z < 23)
    sidx = jnp.stack([jnp.clip(r - 1, 0, NR - 1), r,
                      jnp.clip(r + 1, 0, NR - 1)]).astype(jnp.int32)
    sidx = sidx.reshape(3, NW_, VPW // SCH, SCH).transpose(1, 0, 2, 3)
    sidx = sidx.reshape(NW_, 24, SCH)

    gcols, mcols = [], []
    for dx in (-1, 0, 1):
        for dy in (-1, 0, 1):
            xq, yq = x + dx, y + dy
            oob = (xq < 0) | (xq > 23) | (yq < 0) | (yq > 23)
            gcols.append(jnp.where(oob, r, xq * 576 + yq * 24 + z))
            mcols.append(jnp.where(oob, 0.0, 1.0))
    gidx = jnp.stack(gcols, axis=1).astype(jnp.int32)
    gidx = gidx.reshape(NW_, VPW, 9).transpose(0, 2, 1)
    msk = jnp.stack(mcols, axis=1)
    msk = jnp.pad(msk, ((0, 0), (0, 7)))

    zeros = jnp.zeros((16, 128), jnp.float32)
    z32 = jnp.zeros((M, 32), jnp.float32)
    onem = valid.astype(jnp.float32)[:, None]
    cnt_a = jnp.zeros((M, 32), jnp.float32).at[:, 2].set(1.0)
    cnt_b = jnp.zeros((M, 32), jnp.float32).at[:, 1].set(1.0)
    cnt_c = jnp.zeros((M, 32), jnp.float32).at[:, 0].set(1.0)
    m_a = (ok_a.astype(jnp.float32))[:, None]
    m_b = onem
    m_c = (ok_c.astype(jnp.float32))[:, None]

    def wext(W, NW):
        we = jnp.zeros((9, 128, 64), jnp.float32)
        we = we.at[:, :96, :32].set(W.reshape(9, 96, 32))
        we = we.at[:, 96:99, 32].set(NW.reshape(9, 3))
        hi = we.astype(jnp.bfloat16)
        lo = (we - hi.astype(jnp.float32)).astype(jnp.bfloat16)
        return jnp.concatenate([hi, lo], axis=2)

    def par(gamma, beta):
        p = jnp.zeros((8, 128), jnp.float32)
        p = p.at[0, :32].set(gamma)
        p = p.at[1, :32].set(beta)
        return p

    nv = num_valid_voxels.astype(jnp.int32)

    net = voxel_features.reshape(M, C)
    for we, pp in ((wext(W0, NW0), par(gamma0, beta0)),
                   (wext(W1, NW1), par(gamma1, beta1))):
        upd = jnp.stack([
            jnp.concatenate([z32, z32, net, cnt_a], axis=1) * m_a,
            jnp.concatenate([z32, net, z32, cnt_b], axis=1) * m_b,
            jnp.concatenate([net, z32, z32, cnt_c], axis=1) * m_c,
        ])
        lhs = _SC_KERNEL(upd, sidx, gidx, zeros)
        net = _tc_layer(nv, lhs, msk, we, pp)
    return net.reshape(B, N, C)
